# baseline (speedup 1.0000x reference)
import functools

import jax
import jax.numpy as jnp
import numpy as np
from jax import lax
from jax.experimental import pallas as pl
from jax.experimental.pallas import tpu as pltpu
from jax.experimental.pallas import tpu_sc as plsc

N = 10000
E = 320000
D = 128

NC = 2
NS = 16
NW = NC * NS
EW = E // NW
K = 80
NCHUNK = EW // K
NPAIR = NCHUNK // 2
STRIPE = 624
STRIPE_LAST = N - (NS - 1) * STRIPE


def _worker(c, s):
    return s * NC + c


def _sc_g_body(i0_hbm, i1_hbm, bh_hbm, ch_hbm, g_hbm,
               idx0a, idx1a, idx0b, idx1b,
               bh_a, ch_a, g_a, bh_b, ch_b, g_b,
               sem_a, sem_b, sem_ia, sem_ib, sem_ga, sem_gb):
    wid = _worker(lax.axis_index("c"), lax.axis_index("s"))
    base0 = wid * EW

    def fire_idx(ci, i0_v, i1_v, sem):
        base = base0 + ci * K
        pltpu.async_copy(i0_hbm.at[pl.ds(base, K)], i0_v, sem)
        pltpu.async_copy(i1_hbm.at[pl.ds(base, K)], i1_v, sem)

    def wait_idx(i0_v, i1_v, sem):
        pltpu.make_async_copy(i0_hbm.at[pl.ds(0, K)], i0_v, sem).wait()
        pltpu.make_async_copy(i1_hbm.at[pl.ds(0, K)], i1_v, sem).wait()

    def fire2(i0_v, i1_v, bh_v, ch_v, sem):
        pltpu.async_copy(bh_hbm.at[i0_v], bh_v, sem)
        pltpu.async_copy(ch_hbm.at[i1_v], ch_v, sem)

    def wait2(i0_v, i1_v, bh_v, ch_v, sem):
        pltpu.make_async_copy(bh_hbm.at[i0_v], bh_v, sem).wait()
        pltpu.make_async_copy(ch_hbm.at[i1_v], ch_v, sem).wait()

    def compute(bh_v, ch_v, g_v):
        def row_body(r, rc):
            for cc in range(D // 16):
                sl = pl.ds(cc * 16, 16)
                g_v[r, sl] = bh_v[r, sl] + ch_v[r, sl]
            return rc

        lax.fori_loop(0, K, row_body, 0)

    def fire_g(ci, g_v, sem):
        pltpu.async_copy(g_v, g_hbm.at[pl.ds(base0 + ci * K, K)], sem)

    def wait_g(g_v, sem):
        pltpu.make_async_copy(g_v, g_hbm.at[pl.ds(0, K)], sem).wait()

    pltpu.sync_copy(i0_hbm.at[pl.ds(base0, K)], idx0a)
    pltpu.sync_copy(i1_hbm.at[pl.ds(base0, K)], idx1a)
    fire2(idx0a, idx1a, bh_a, ch_a, sem_a)
    fire_idx(1, idx0b, idx1b, sem_ib)

    def pair_body(pi, carry):
        c0 = 2 * pi
        c1 = c0 + 1
        wait2(idx0a, idx1a, bh_a, ch_a, sem_a)
        wait_idx(idx0b, idx1b, sem_ib)

        @pl.when(pi > 0)
        def _():
            wait_g(g_b, sem_gb)

        fire2(idx0b, idx1b, bh_b, ch_b, sem_b)
        compute(bh_a, ch_a, g_a)
        fire_g(c0, g_a, sem_ga)
        fire_idx(c0 + 2, idx0a, idx1a, sem_ia)
        wait2(idx0b, idx1b, bh_b, ch_b, sem_b)
        wait_idx(idx0a, idx1a, sem_ia)
        wait_g(g_a, sem_ga)

        @pl.when(pi < NPAIR - 1)
        def _():
            fire2(idx0a, idx1a, bh_a, ch_a, sem_a)

        compute(bh_b, ch_b, g_b)
        fire_g(c1, g_b, sem_gb)
        fire_idx(c0 + 3, idx0b, idx1b, sem_ib)
        return carry

    lax.fori_loop(0, NPAIR, pair_body, 0)
    wait_idx(idx0b, idx1b, sem_ib)
    wait_g(g_b, sem_gb)
    if NCHUNK % 2:
        ci = NCHUNK - 1
        base = base0 + ci * K
        pltpu.sync_copy(i0_hbm.at[pl.ds(base, K)], idx0a)
        pltpu.sync_copy(i1_hbm.at[pl.ds(base, K)], idx1a)
        pltpu.sync_copy(bh_hbm.at[idx0a], bh_a)
        pltpu.sync_copy(ch_hbm.at[idx1a], ch_a)
        compute(bh_a, ch_a, g_a)
        pltpu.sync_copy(g_a, g_hbm.at[pl.ds(base, K)])


def _sc_g(idx0, idx1, bh, ch):
    mesh = plsc.VectorSubcoreMesh(core_axis_name="c", subcore_axis_name="s",
                                  num_cores=NC, num_subcores=NS)
    return pl.kernel(
        _sc_g_body,
        out_type=jax.ShapeDtypeStruct((E, D), jnp.float32),
        mesh=mesh,
        scratch_types=(
            [pltpu.VMEM((K,), jnp.int32)] * 4
            + [pltpu.VMEM((K, D), jnp.float32)] * 6
            + [pltpu.SemaphoreType.DMA] * 6
        ),
    )(idx0, idx1, bh, ch)


def _sc_msg_body(e_hbm, i0_hbm, i1_hbm, vh_hbm, zeros_hbm, agg_hbm,
                 idx0a, idx1a, idx0b, idx1b,
                 e_a, vh_a, e_b, vh_b,
                 agg_sh, sem_a, sem_b, sem_ia, sem_ib):
    c = lax.axis_index("c")
    s = lax.axis_index("s")
    wid = _worker(c, s)
    base0 = wid * EW

    @pl.when(s < NS - 1)
    def _():
        pltpu.sync_copy(zeros_hbm.at[pl.ds(s * STRIPE, STRIPE)],
                        agg_sh.at[pl.ds(s * STRIPE, STRIPE)])

    @pl.when(s == NS - 1)
    def _():
        pltpu.sync_copy(zeros_hbm.at[pl.ds((NS - 1) * STRIPE, STRIPE_LAST)],
                        agg_sh.at[pl.ds((NS - 1) * STRIPE, STRIPE_LAST)])

    plsc.subcore_barrier()

    def fire_idx(ci, i0_v, i1_v, sem):
        base = base0 + ci * K
        pltpu.async_copy(i0_hbm.at[pl.ds(base, K)], i0_v, sem)
        pltpu.async_copy(i1_hbm.at[pl.ds(base, K)], i1_v, sem)

    def wait_idx(i0_v, i1_v, sem):
        pltpu.make_async_copy(i0_hbm.at[pl.ds(0, K)], i0_v, sem).wait()
        pltpu.make_async_copy(i1_hbm.at[pl.ds(0, K)], i1_v, sem).wait()

    def fire2(ci, i1_v, e_v, vh_v, sem):
        pltpu.async_copy(vh_hbm.at[i1_v], vh_v, sem)
        pltpu.async_copy(e_hbm.at[pl.ds(base0 + ci * K, K)], e_v, sem)

    def wait2(i1_v, e_v, vh_v, sem):
        pltpu.make_async_copy(vh_hbm.at[i1_v], vh_v, sem).wait()
        pltpu.make_async_copy(e_hbm.at[pl.ds(0, K)], e_v, sem).wait()

    def compute(e_v, vh_v):
        def row_body(r, rc):
            for cc in range(D // 16):
                sl = pl.ds(cc * 16, 16)
                x = e_v[r, sl]
                e_v[r, sl] = vh_v[r, sl] / (1.0 + jnp.exp(-x))
            return rc

        lax.fori_loop(0, K, row_body, 0)

    pltpu.sync_copy(i0_hbm.at[pl.ds(base0, K)], idx0a)
    pltpu.sync_copy(i1_hbm.at[pl.ds(base0, K)], idx1a)
    fire2(0, idx1a, e_a, vh_a, sem_a)
    fire_idx(1, idx0b, idx1b, sem_ib)

    def pair_body(pi, carry):
        c0 = 2 * pi
        c1 = c0 + 1
        wait2(idx1a, e_a, vh_a, sem_a)
        wait_idx(idx0b, idx1b, sem_ib)
        fire2(c1, idx1b, e_b, vh_b, sem_b)
        compute(e_a, vh_a)
        pltpu.sync_copy(e_a, agg_sh.at[idx0a], add=True)
        fire_idx(c0 + 2, idx0a, idx1a, sem_ia)
        wait2(idx1b, e_b, vh_b, sem_b)
        wait_idx(idx0a, idx1a, sem_ia)

        @pl.when(pi < NPAIR - 1)
        def _():
            fire2(c0 + 2, idx1a, e_a, vh_a, sem_a)

        compute(e_b, vh_b)
        pltpu.sync_copy(e_b, agg_sh.at[idx0b], add=True)
        fire_idx(c0 + 3, idx0b, idx1b, sem_ib)
        return carry

    lax.fori_loop(0, NPAIR, pair_body, 0)
    wait_idx(idx0b, idx1b, sem_ib)
    if NCHUNK % 2:
        ci = NCHUNK - 1
        base = base0 + ci * K
        pltpu.sync_copy(i0_hbm.at[pl.ds(base, K)], idx0a)
        pltpu.sync_copy(i1_hbm.at[pl.ds(base, K)], idx1a)
        pltpu.sync_copy(vh_hbm.at[idx1a], vh_a)
        pltpu.sync_copy(e_hbm.at[pl.ds(base, K)], e_a)
        compute(e_a, vh_a)
        pltpu.sync_copy(e_a, agg_sh.at[idx0a], add=True)
    plsc.subcore_barrier()

    @pl.when(s < NS - 1)
    def _():
        pltpu.sync_copy(agg_sh.at[pl.ds(s * STRIPE, STRIPE)],
                        agg_hbm.at[c, pl.ds(s * STRIPE, STRIPE)])

    @pl.when(s == NS - 1)
    def _():
        pltpu.sync_copy(agg_sh.at[pl.ds((NS - 1) * STRIPE, STRIPE_LAST)],
                        agg_hbm.at[c, pl.ds((NS - 1) * STRIPE, STRIPE_LAST)])


def _sc_msg(e_in, idx0, idx1, vh, zeros_n):
    mesh = plsc.VectorSubcoreMesh(core_axis_name="c", subcore_axis_name="s",
                                  num_cores=NC, num_subcores=NS)
    return pl.kernel(
        _sc_msg_body,
        out_type=jax.ShapeDtypeStruct((NC, N, D), jnp.float32),
        mesh=mesh,
        compiler_params=pltpu.CompilerParams(skip_device_barrier=True),
        scratch_types=(
            [pltpu.VMEM((K,), jnp.int32)] * 4
            + [pltpu.VMEM((K, D), jnp.float32)] * 4
            + [pltpu.VMEM_SHARED((N, D), jnp.float32)]
            + [pltpu.SemaphoreType.DMA] * 4
        ),
    )(e_in, idx0, idx1, vh, zeros_n)


_NB = 1000
_EB = 2000


def _tables_body(h_ref, uw, ub, vw, vb, bw, bb, cw, cb,
                 uh_ref, vh_ref, bh_ref, ch_ref):
    h = h_ref[...]
    uh_ref[...] = jnp.dot(h, uw[...], preferred_element_type=jnp.float32) + ub[...]
    vh_ref[...] = jnp.dot(h, vw[...], preferred_element_type=jnp.float32) + vb[...]
    bh_ref[...] = jnp.dot(h, bw[...], preferred_element_type=jnp.float32) + bb[...]
    ch_ref[...] = jnp.dot(h, cw[...], preferred_element_type=jnp.float32) + cb[...]


def _tables(h_in, uwt, ub, vwt, vb, bwt, bb, cwt, cb):
    w_spec = pl.BlockSpec((D, D), lambda i: (0, 0))
    b_spec = pl.BlockSpec((1, D), lambda i: (0, 0))
    x_spec = pl.BlockSpec((_NB, D), lambda i: (i, 0))
    return pl.pallas_call(
        _tables_body,
        grid=(N // _NB,),
        in_specs=[x_spec, w_spec, b_spec, w_spec, b_spec, w_spec, b_spec,
                  w_spec, b_spec],
        out_specs=[x_spec, x_spec, x_spec, x_spec],
        out_shape=[jax.ShapeDtypeStruct((N, D), jnp.float32)] * 4,
    )(h_in, uwt, ub, vwt, vb, bwt, bb, cwt, cb)


def _estats_body(e_ref, at, ab, g_ref, sum_ref, ssq_ref):
    pre = (jnp.dot(e_ref[...], at[...], preferred_element_type=jnp.float32)
           + ab[...] + g_ref[...])

    @pl.when(pl.program_id(0) == 0)
    def _():
        sum_ref[...] = jnp.zeros_like(sum_ref)
        ssq_ref[...] = jnp.zeros_like(ssq_ref)

    sum_ref[...] += jnp.sum(pre, axis=0, keepdims=True)
    ssq_ref[...] += jnp.sum(pre * pre, axis=0, keepdims=True)


def _estats(e_in, awt, ab, g):
    w_spec = pl.BlockSpec((D, D), lambda i: (0, 0))
    b_spec = pl.BlockSpec((1, D), lambda i: (0, 0))
    x_spec = pl.BlockSpec((_EB, D), lambda i: (i, 0))
    return pl.pallas_call(
        _estats_body,
        grid=(E // _EB,),
        in_specs=[x_spec, w_spec, b_spec, x_spec],
        out_specs=[b_spec, b_spec],
        out_shape=[jax.ShapeDtypeStruct((1, D), jnp.float32)] * 2,
    )(e_in, awt, ab, g)


def _eout_body(e_ref, at, ab, g_ref, sum_ref, ssq_ref, gam, bet, out_ref):
    pre = (jnp.dot(e_ref[...], at[...], preferred_element_type=jnp.float32)
           + ab[...] + g_ref[...])
    mean = sum_ref[...] * (1.0 / E)
    var = ssq_ref[...] * (1.0 / E) - mean * mean
    inv = lax.rsqrt(var + 1e-5)
    bn = (pre - mean) * inv * gam[...] + bet[...]
    out_ref[...] = e_ref[...] + jnp.maximum(bn, 0.0)


def _eout(e_in, awt, ab, g, esum, essq, gam, bet):
    w_spec = pl.BlockSpec((D, D), lambda i: (0, 0))
    b_spec = pl.BlockSpec((1, D), lambda i: (0, 0))
    x_spec = pl.BlockSpec((_EB, D), lambda i: (i, 0))
    return pl.pallas_call(
        _eout_body,
        grid=(E // _EB,),
        in_specs=[x_spec, w_spec, b_spec, x_spec, b_spec, b_spec, b_spec,
                  b_spec],
        out_specs=x_spec,
        out_shape=jax.ShapeDtypeStruct((E, D), jnp.float32),
    )(e_in, awt, ab, g, esum, essq, gam, bet)


def _hstats_body(uh_ref, a0_ref, a1_ref, sum_ref, ssq_ref):
    pre = uh_ref[...] + a0_ref[...] + a1_ref[...]

    @pl.when(pl.program_id(0) == 0)
    def _():
        sum_ref[...] = jnp.zeros_like(sum_ref)
        ssq_ref[...] = jnp.zeros_like(ssq_ref)

    sum_ref[...] += jnp.sum(pre, axis=0, keepdims=True)
    ssq_ref[...] += jnp.sum(pre * pre, axis=0, keepdims=True)


def _hstats(uh, a0, a1):
    b_spec = pl.BlockSpec((1, D), lambda i: (0, 0))
    x_spec = pl.BlockSpec((_NB, D), lambda i: (i, 0))
    return pl.pallas_call(
        _hstats_body,
        grid=(N // _NB,),
        in_specs=[x_spec, x_spec, x_spec],
        out_specs=[b_spec, b_spec],
        out_shape=[jax.ShapeDtypeStruct((1, D), jnp.float32)] * 2,
    )(uh, a0, a1)


def _hout_body(h_ref, uh_ref, a0_ref, a1_ref, sum_ref, ssq_ref, gam, bet,
               out_ref):
    pre = uh_ref[...] + a0_ref[...] + a1_ref[...]
    mean = sum_ref[...] * (1.0 / N)
    var = ssq_ref[...] * (1.0 / N) - mean * mean
    inv = lax.rsqrt(var + 1e-5)
    bn = (pre - mean) * inv * gam[...] + bet[...]
    out_ref[...] = h_ref[...] + jnp.maximum(bn, 0.0)


def _hout(h_in, uh, a0, a1, hsum, hssq, gam, bet):
    b_spec = pl.BlockSpec((1, D), lambda i: (0, 0))
    x_spec = pl.BlockSpec((_NB, D), lambda i: (i, 0))
    return pl.pallas_call(
        _hout_body,
        grid=(N // _NB,),
        in_specs=[x_spec, x_spec, x_spec, x_spec, b_spec, b_spec, b_spec,
                  b_spec],
        out_specs=x_spec,
        out_shape=jax.ShapeDtypeStruct((N, D), jnp.float32),
    )(h_in, uh, a0, a1, hsum, hssq, gam, bet)


def kernel(h_in, e_in, edge_index, U_w, U_b, V_w, V_b, A_w, A_b, B_w, B_b,
           C_w, C_b, gamma_h, beta_h, gamma_e, beta_e):
    pad = jnp.zeros((2 * K,), jnp.int32)
    idx0 = jnp.concatenate([edge_index[0].astype(jnp.int32), pad])
    idx1 = jnp.concatenate([edge_index[1].astype(jnp.int32), pad])
    ub = U_b.reshape(1, D)
    vb = V_b.reshape(1, D)
    ab = A_b.reshape(1, D)
    bb = B_b.reshape(1, D)
    cb = C_b.reshape(1, D)
    gh = gamma_h.reshape(1, D)
    bh_ = beta_h.reshape(1, D)
    ge = gamma_e.reshape(1, D)
    be = beta_e.reshape(1, D)

    uh, vh, bh, ch = _tables(h_in, U_w.T, ub, V_w.T, vb, B_w.T, bb, C_w.T, cb)

    zeros_n = jnp.zeros((N, D), jnp.float32)
    g = _sc_g(idx0, idx1, bh, ch)
    agg = _sc_msg(e_in, idx0, idx1, vh, zeros_n)

    esum, essq = _estats(e_in, A_w.T, ab, g)
    e_out = _eout(e_in, A_w.T, ab, g, esum, essq, ge, be)

    hsum, hssq = _hstats(uh, agg[0], agg[1])
    h_out = _hout(h_in, uh, agg[0], agg[1], hsum, hssq, gh, bh_)
    return (h_out, e_out)


# speedup vs baseline: 1.1083x; 1.1083x over previous
import functools

import jax
import jax.numpy as jnp
import numpy as np
from jax import lax
from jax.experimental import pallas as pl
from jax.experimental.pallas import tpu as pltpu
from jax.experimental.pallas import tpu_sc as plsc

N = 10000
E = 320000
D = 128

NC = 2
NS = 16
NW = NC * NS
EW = E // NW
K = 40
NCHUNK = EW // K
NPAIR = NCHUNK // 2
STRIPE = 624
STRIPE_LAST = N - (NS - 1) * STRIPE


def _worker(c, s):
    return s * NC + c


def _sc_edge_body(e_hbm, i0_hbm, i1_hbm, vh_hbm, bh_hbm, ch_hbm, zeros_hbm,
                  g_hbm, agg_hbm,
                  idx0a, idx1a, idx0b, idx1b,
                  e_a, vh_a, bh_a, ch_a, e_b, vh_b, bh_b, ch_b,
                  agg_sh, sem_a, sem_b, sem_ia, sem_ib, sem_ga, sem_gb):
    c = lax.axis_index("c")
    s = lax.axis_index("s")
    wid = _worker(c, s)
    base0 = wid * EW

    @pl.when(s < NS - 1)
    def _():
        pltpu.sync_copy(zeros_hbm.at[pl.ds(s * STRIPE, STRIPE)],
                        agg_sh.at[pl.ds(s * STRIPE, STRIPE)])

    @pl.when(s == NS - 1)
    def _():
        pltpu.sync_copy(zeros_hbm.at[pl.ds((NS - 1) * STRIPE, STRIPE_LAST)],
                        agg_sh.at[pl.ds((NS - 1) * STRIPE, STRIPE_LAST)])

    plsc.subcore_barrier()

    def fire_idx(ci, i0_v, i1_v, sem):
        base = base0 + ci * K
        pltpu.async_copy(i0_hbm.at[pl.ds(base, K)], i0_v, sem)
        pltpu.async_copy(i1_hbm.at[pl.ds(base, K)], i1_v, sem)

    def wait_idx(i0_v, i1_v, sem):
        pltpu.make_async_copy(i0_hbm.at[pl.ds(0, K)], i0_v, sem).wait()
        pltpu.make_async_copy(i1_hbm.at[pl.ds(0, K)], i1_v, sem).wait()

    def fire4(ci, i0_v, i1_v, e_v, vh_v, bh_v, ch_v, sem):
        pltpu.async_copy(vh_hbm.at[i1_v], vh_v, sem)
        pltpu.async_copy(bh_hbm.at[i0_v], bh_v, sem)
        pltpu.async_copy(ch_hbm.at[i1_v], ch_v, sem)
        pltpu.async_copy(e_hbm.at[pl.ds(base0 + ci * K, K)], e_v, sem)

    def wait4(i0_v, i1_v, e_v, vh_v, bh_v, ch_v, sem):
        pltpu.make_async_copy(vh_hbm.at[i1_v], vh_v, sem).wait()
        pltpu.make_async_copy(bh_hbm.at[i0_v], bh_v, sem).wait()
        pltpu.make_async_copy(ch_hbm.at[i1_v], ch_v, sem).wait()
        pltpu.make_async_copy(e_hbm.at[pl.ds(0, K)], e_v, sem).wait()

    def compute(e_v, vh_v, bh_v, ch_v):
        def row_body(r, rc):
            for cc in range(D // 16):
                sl = pl.ds(cc * 16, 16)
                x = e_v[r, sl]
                e_v[r, sl] = vh_v[r, sl] / (1.0 + jnp.exp(-x))
                vh_v[r, sl] = bh_v[r, sl] + ch_v[r, sl]
            return rc

        lax.fori_loop(0, K, row_body, 0)

    def fire_g(ci, g_v, sem):
        pltpu.async_copy(g_v, g_hbm.at[pl.ds(base0 + ci * K, K)], sem)

    def wait_g(g_v, sem):
        pltpu.make_async_copy(g_v, g_hbm.at[pl.ds(0, K)], sem).wait()

    pltpu.sync_copy(i0_hbm.at[pl.ds(base0, K)], idx0a)
    pltpu.sync_copy(i1_hbm.at[pl.ds(base0, K)], idx1a)
    fire4(0, idx0a, idx1a, e_a, vh_a, bh_a, ch_a, sem_a)
    fire_idx(1, idx0b, idx1b, sem_ib)

    def pair_body(pi, carry):
        c0 = 2 * pi
        c1 = c0 + 1
        wait4(idx0a, idx1a, e_a, vh_a, bh_a, ch_a, sem_a)
        wait_idx(idx0b, idx1b, sem_ib)

        @pl.when(pi > 0)
        def _():
            wait_g(vh_b, sem_gb)

        fire4(c1, idx0b, idx1b, e_b, vh_b, bh_b, ch_b, sem_b)
        compute(e_a, vh_a, bh_a, ch_a)
        fire_g(c0, vh_a, sem_ga)
        pltpu.sync_copy(e_a, agg_sh.at[idx0a], add=True)
        fire_idx(c0 + 2, idx0a, idx1a, sem_ia)
        wait4(idx0b, idx1b, e_b, vh_b, bh_b, ch_b, sem_b)
        wait_idx(idx0a, idx1a, sem_ia)
        wait_g(vh_a, sem_ga)

        @pl.when(pi < NPAIR - 1)
        def _():
            fire4(c0 + 2, idx0a, idx1a, e_a, vh_a, bh_a, ch_a, sem_a)

        compute(e_b, vh_b, bh_b, ch_b)
        fire_g(c1, vh_b, sem_gb)
        pltpu.sync_copy(e_b, agg_sh.at[idx0b], add=True)
        fire_idx(c0 + 3, idx0b, idx1b, sem_ib)
        return carry

    lax.fori_loop(0, NPAIR, pair_body, 0)
    wait_idx(idx0b, idx1b, sem_ib)
    wait_g(vh_b, sem_gb)
    plsc.subcore_barrier()

    @pl.when(s < NS - 1)
    def _():
        pltpu.sync_copy(agg_sh.at[pl.ds(s * STRIPE, STRIPE)],
                        agg_hbm.at[c, pl.ds(s * STRIPE, STRIPE)])

    @pl.when(s == NS - 1)
    def _():
        pltpu.sync_copy(agg_sh.at[pl.ds((NS - 1) * STRIPE, STRIPE_LAST)],
                        agg_hbm.at[c, pl.ds((NS - 1) * STRIPE, STRIPE_LAST)])


def _sc_edges(e_in, idx0, idx1, vh, bh, ch, zeros_n):
    mesh = plsc.VectorSubcoreMesh(core_axis_name="c", subcore_axis_name="s",
                                  num_cores=NC, num_subcores=NS)
    return pl.kernel(
        _sc_edge_body,
        out_type=(jax.ShapeDtypeStruct((E, D), jnp.float32),
                  jax.ShapeDtypeStruct((NC, N, D), jnp.float32)),
        mesh=mesh,
        scratch_types=(
            [pltpu.VMEM((K,), jnp.int32)] * 4
            + [pltpu.VMEM((K, D), jnp.float32)] * 8
            + [pltpu.VMEM_SHARED((N, D), jnp.float32)]
            + [pltpu.SemaphoreType.DMA] * 6
        ),
    )(e_in, idx0, idx1, vh, bh, ch, zeros_n)


_NB = 1000
_EB = 2000


def _tables_body(h_ref, uw, ub, vw, vb, bw, bb, cw, cb,
                 uh_ref, vh_ref, bh_ref, ch_ref):
    h = h_ref[...]
    uh_ref[...] = jnp.dot(h, uw[...], preferred_element_type=jnp.float32) + ub[...]
    vh_ref[...] = jnp.dot(h, vw[...], preferred_element_type=jnp.float32) + vb[...]
    bh_ref[...] = jnp.dot(h, bw[...], preferred_element_type=jnp.float32) + bb[...]
    ch_ref[...] = jnp.dot(h, cw[...], preferred_element_type=jnp.float32) + cb[...]


def _tables(h_in, uwt, ub, vwt, vb, bwt, bb, cwt, cb):
    w_spec = pl.BlockSpec((D, D), lambda i: (0, 0))
    b_spec = pl.BlockSpec((1, D), lambda i: (0, 0))
    x_spec = pl.BlockSpec((_NB, D), lambda i: (i, 0))
    return pl.pallas_call(
        _tables_body,
        grid=(N // _NB,),
        in_specs=[x_spec, w_spec, b_spec, w_spec, b_spec, w_spec, b_spec,
                  w_spec, b_spec],
        out_specs=[x_spec, x_spec, x_spec, x_spec],
        out_shape=[jax.ShapeDtypeStruct((N, D), jnp.float32)] * 4,
    )(h_in, uwt, ub, vwt, vb, bwt, bb, cwt, cb)


def _estats_body(e_ref, at, ab, g_ref, sum_ref, ssq_ref):
    pre = (jnp.dot(e_ref[...], at[...], preferred_element_type=jnp.float32)
           + ab[...] + g_ref[...])

    @pl.when(pl.program_id(0) == 0)
    def _():
        sum_ref[...] = jnp.zeros_like(sum_ref)
        ssq_ref[...] = jnp.zeros_like(ssq_ref)

    sum_ref[...] += jnp.sum(pre, axis=0, keepdims=True)
    ssq_ref[...] += jnp.sum(pre * pre, axis=0, keepdims=True)


def _estats(e_in, awt, ab, g):
    w_spec = pl.BlockSpec((D, D), lambda i: (0, 0))
    b_spec = pl.BlockSpec((1, D), lambda i: (0, 0))
    x_spec = pl.BlockSpec((_EB, D), lambda i: (i, 0))
    return pl.pallas_call(
        _estats_body,
        grid=(E // _EB,),
        in_specs=[x_spec, w_spec, b_spec, x_spec],
        out_specs=[b_spec, b_spec],
        out_shape=[jax.ShapeDtypeStruct((1, D), jnp.float32)] * 2,
    )(e_in, awt, ab, g)


def _eout_body(e_ref, at, ab, g_ref, sum_ref, ssq_ref, gam, bet, out_ref):
    pre = (jnp.dot(e_ref[...], at[...], preferred_element_type=jnp.float32)
           + ab[...] + g_ref[...])
    mean = sum_ref[...] * (1.0 / E)
    var = ssq_ref[...] * (1.0 / E) - mean * mean
    inv = lax.rsqrt(var + 1e-5)
    bn = (pre - mean) * inv * gam[...] + bet[...]
    out_ref[...] = e_ref[...] + jnp.maximum(bn, 0.0)


def _eout(e_in, awt, ab, g, esum, essq, gam, bet):
    w_spec = pl.BlockSpec((D, D), lambda i: (0, 0))
    b_spec = pl.BlockSpec((1, D), lambda i: (0, 0))
    x_spec = pl.BlockSpec((_EB, D), lambda i: (i, 0))
    return pl.pallas_call(
        _eout_body,
        grid=(E // _EB,),
        in_specs=[x_spec, w_spec, b_spec, x_spec, b_spec, b_spec, b_spec,
                  b_spec],
        out_specs=x_spec,
        out_shape=jax.ShapeDtypeStruct((E, D), jnp.float32),
    )(e_in, awt, ab, g, esum, essq, gam, bet)


def _hstats_body(uh_ref, a0_ref, a1_ref, sum_ref, ssq_ref):
    pre = uh_ref[...] + a0_ref[...] + a1_ref[...]

    @pl.when(pl.program_id(0) == 0)
    def _():
        sum_ref[...] = jnp.zeros_like(sum_ref)
        ssq_ref[...] = jnp.zeros_like(ssq_ref)

    sum_ref[...] += jnp.sum(pre, axis=0, keepdims=True)
    ssq_ref[...] += jnp.sum(pre * pre, axis=0, keepdims=True)


def _hstats(uh, a0, a1):
    b_spec = pl.BlockSpec((1, D), lambda i: (0, 0))
    x_spec = pl.BlockSpec((_NB, D), lambda i: (i, 0))
    return pl.pallas_call(
        _hstats_body,
        grid=(N // _NB,),
        in_specs=[x_spec, x_spec, x_spec],
        out_specs=[b_spec, b_spec],
        out_shape=[jax.ShapeDtypeStruct((1, D), jnp.float32)] * 2,
    )(uh, a0, a1)


def _hout_body(h_ref, uh_ref, a0_ref, a1_ref, sum_ref, ssq_ref, gam, bet,
               out_ref):
    pre = uh_ref[...] + a0_ref[...] + a1_ref[...]
    mean = sum_ref[...] * (1.0 / N)
    var = ssq_ref[...] * (1.0 / N) - mean * mean
    inv = lax.rsqrt(var + 1e-5)
    bn = (pre - mean) * inv * gam[...] + bet[...]
    out_ref[...] = h_ref[...] + jnp.maximum(bn, 0.0)


def _hout(h_in, uh, a0, a1, hsum, hssq, gam, bet):
    b_spec = pl.BlockSpec((1, D), lambda i: (0, 0))
    x_spec = pl.BlockSpec((_NB, D), lambda i: (i, 0))
    return pl.pallas_call(
        _hout_body,
        grid=(N // _NB,),
        in_specs=[x_spec, x_spec, x_spec, x_spec, b_spec, b_spec, b_spec,
                  b_spec],
        out_specs=x_spec,
        out_shape=jax.ShapeDtypeStruct((N, D), jnp.float32),
    )(h_in, uh, a0, a1, hsum, hssq, gam, bet)


def kernel(h_in, e_in, edge_index, U_w, U_b, V_w, V_b, A_w, A_b, B_w, B_b,
           C_w, C_b, gamma_h, beta_h, gamma_e, beta_e):
    pad = jnp.zeros((2 * K,), jnp.int32)
    idx0 = jnp.concatenate([edge_index[0].astype(jnp.int32), pad])
    idx1 = jnp.concatenate([edge_index[1].astype(jnp.int32), pad])
    ub = U_b.reshape(1, D)
    vb = V_b.reshape(1, D)
    ab = A_b.reshape(1, D)
    bb = B_b.reshape(1, D)
    cb = C_b.reshape(1, D)
    gh = gamma_h.reshape(1, D)
    bh_ = beta_h.reshape(1, D)
    ge = gamma_e.reshape(1, D)
    be = beta_e.reshape(1, D)

    uh, vh, bh, ch = _tables(h_in, U_w.T, ub, V_w.T, vb, B_w.T, bb, C_w.T, cb)

    zeros_n = jnp.zeros((N, D), jnp.float32)
    g, agg = _sc_edges(e_in, idx0, idx1, vh, bh, ch, zeros_n)

    esum, essq = _estats(e_in, A_w.T, ab, g)
    e_out = _eout(e_in, A_w.T, ab, g, esum, essq, ge, be)

    hsum, hssq = _hstats(uh, agg[0], agg[1])
    h_out = _hout(h_in, uh, agg[0], agg[1], hsum, hssq, gh, bh_)
    return (h_out, e_out)


# speedup vs baseline: 1.2431x; 1.1217x over previous
import functools

import jax
import jax.numpy as jnp
import numpy as np
from jax import lax
from jax.experimental import pallas as pl
from jax.experimental.pallas import tpu as pltpu
from jax.experimental.pallas import tpu_sc as plsc

N = 10000
E = 320000
D = 128

NC = 2
NS = 16
NW = NC * NS
EW = E // NW
K = 40
NCHUNK = EW // K
NPAIR = NCHUNK // 2
STRIPE = 624
STRIPE_LAST = N - (NS - 1) * STRIPE


def _worker(c, s):
    return s * NC + c


def _sc_edge_body(e_hbm, i0_hbm, i1_hbm, vh_hbm, bh_hbm, ch_hbm, zeros_hbm,
                  g_hbm, agg_hbm,
                  idx0a, idx1a, idx0b, idx1b,
                  e_a, vh_a, bh_a, ch_a, e_b, vh_b, bh_b, ch_b,
                  agg_sh, sem_a, sem_b, sem_ia, sem_ib, sem_ga, sem_gb):
    c = lax.axis_index("c")
    s = lax.axis_index("s")
    wid = _worker(c, s)
    base0 = wid * EW

    @pl.when(s < NS - 1)
    def _():
        pltpu.sync_copy(zeros_hbm.at[pl.ds(s * STRIPE, STRIPE)],
                        agg_sh.at[pl.ds(s * STRIPE, STRIPE)])

    @pl.when(s == NS - 1)
    def _():
        pltpu.sync_copy(zeros_hbm.at[pl.ds((NS - 1) * STRIPE, STRIPE_LAST)],
                        agg_sh.at[pl.ds((NS - 1) * STRIPE, STRIPE_LAST)])

    plsc.subcore_barrier()

    def fire_idx(ci, i0_v, i1_v, sem):
        base = base0 + ci * K
        pltpu.async_copy(i0_hbm.at[pl.ds(base, K)], i0_v, sem)
        pltpu.async_copy(i1_hbm.at[pl.ds(base, K)], i1_v, sem)

    def wait_idx(i0_v, i1_v, sem):
        pltpu.make_async_copy(i0_hbm.at[pl.ds(0, K)], i0_v, sem).wait()
        pltpu.make_async_copy(i1_hbm.at[pl.ds(0, K)], i1_v, sem).wait()

    def fire4(ci, i0_v, i1_v, e_v, vh_v, bh_v, ch_v, sem):
        pltpu.async_copy(vh_hbm.at[i1_v], vh_v, sem)
        pltpu.async_copy(bh_hbm.at[i0_v], bh_v, sem)
        pltpu.async_copy(ch_hbm.at[i1_v], ch_v, sem)
        pltpu.async_copy(e_hbm.at[pl.ds(base0 + ci * K, K)], e_v, sem)

    def wait4(i0_v, i1_v, e_v, vh_v, bh_v, ch_v, sem):
        pltpu.make_async_copy(vh_hbm.at[i1_v], vh_v, sem).wait()
        pltpu.make_async_copy(bh_hbm.at[i0_v], bh_v, sem).wait()
        pltpu.make_async_copy(ch_hbm.at[i1_v], ch_v, sem).wait()
        pltpu.make_async_copy(e_hbm.at[pl.ds(0, K)], e_v, sem).wait()

    def compute(e_v, vh_v, bh_v, ch_v):
        def row_body(r, rc):
            for cc in range(D // 16):
                sl = pl.ds(cc * 16, 16)
                x = e_v[r, sl]
                e_v[r, sl] = vh_v[r, sl] / (1.0 + jnp.exp(-x))
                vh_v[r, sl] = bh_v[r, sl] + ch_v[r, sl]
            return rc

        lax.fori_loop(0, K, row_body, 0)

    def fire_g(ci, g_v, sem):
        pltpu.async_copy(g_v, g_hbm.at[pl.ds(base0 + ci * K, K)], sem)

    def wait_g(g_v, sem):
        pltpu.make_async_copy(g_v, g_hbm.at[pl.ds(0, K)], sem).wait()

    pltpu.sync_copy(i0_hbm.at[pl.ds(base0, K)], idx0a)
    pltpu.sync_copy(i1_hbm.at[pl.ds(base0, K)], idx1a)
    fire4(0, idx0a, idx1a, e_a, vh_a, bh_a, ch_a, sem_a)
    fire_idx(1, idx0b, idx1b, sem_ib)

    def pair_body(pi, carry):
        c0 = 2 * pi
        c1 = c0 + 1
        wait4(idx0a, idx1a, e_a, vh_a, bh_a, ch_a, sem_a)
        wait_idx(idx0b, idx1b, sem_ib)

        @pl.when(pi > 0)
        def _():
            wait_g(vh_b, sem_gb)

        fire4(c1, idx0b, idx1b, e_b, vh_b, bh_b, ch_b, sem_b)
        compute(e_a, vh_a, bh_a, ch_a)
        fire_g(c0, vh_a, sem_ga)
        pltpu.sync_copy(e_a, agg_sh.at[idx0a], add=True)
        fire_idx(c0 + 2, idx0a, idx1a, sem_ia)
        wait4(idx0b, idx1b, e_b, vh_b, bh_b, ch_b, sem_b)
        wait_idx(idx0a, idx1a, sem_ia)
        wait_g(vh_a, sem_ga)

        @pl.when(pi < NPAIR - 1)
        def _():
            fire4(c0 + 2, idx0a, idx1a, e_a, vh_a, bh_a, ch_a, sem_a)

        compute(e_b, vh_b, bh_b, ch_b)
        fire_g(c1, vh_b, sem_gb)
        pltpu.sync_copy(e_b, agg_sh.at[idx0b], add=True)
        fire_idx(c0 + 3, idx0b, idx1b, sem_ib)
        return carry

    lax.fori_loop(0, NPAIR, pair_body, 0)
    wait_idx(idx0b, idx1b, sem_ib)
    wait_g(vh_b, sem_gb)
    plsc.subcore_barrier()

    @pl.when(s < NS - 1)
    def _():
        pltpu.sync_copy(agg_sh.at[pl.ds(s * STRIPE, STRIPE)],
                        agg_hbm.at[c, pl.ds(s * STRIPE, STRIPE)])

    @pl.when(s == NS - 1)
    def _():
        pltpu.sync_copy(agg_sh.at[pl.ds((NS - 1) * STRIPE, STRIPE_LAST)],
                        agg_hbm.at[c, pl.ds((NS - 1) * STRIPE, STRIPE_LAST)])


def _sc_edges(e_in, idx0, idx1, vh, bh, ch, zeros_n):
    mesh = plsc.VectorSubcoreMesh(core_axis_name="c", subcore_axis_name="s",
                                  num_cores=NC, num_subcores=NS)
    return pl.kernel(
        _sc_edge_body,
        out_type=(jax.ShapeDtypeStruct((E, D), jnp.float32),
                  jax.ShapeDtypeStruct((NC, N, D), jnp.float32)),
        mesh=mesh,
        scratch_types=(
            [pltpu.VMEM((K,), jnp.int32)] * 4
            + [pltpu.VMEM((K, D), jnp.float32)] * 8
            + [pltpu.VMEM_SHARED((N, D), jnp.float32)]
            + [pltpu.SemaphoreType.DMA] * 6
        ),
    )(e_in, idx0, idx1, vh, bh, ch, zeros_n)


_NB = 1000
_EB = 4000


def _tables_body(h_ref, uw, ub, vw, vb, bw, bb, cw, cb,
                 uh_ref, vh_ref, bh_ref, ch_ref):
    h = h_ref[...]
    uh_ref[...] = jnp.dot(h, uw[...], preferred_element_type=jnp.float32) + ub[...]
    vh_ref[...] = jnp.dot(h, vw[...], preferred_element_type=jnp.float32) + vb[...]
    bh_ref[...] = jnp.dot(h, bw[...], preferred_element_type=jnp.float32) + bb[...]
    ch_ref[...] = jnp.dot(h, cw[...], preferred_element_type=jnp.float32) + cb[...]


def _tables(h_in, uwt, ub, vwt, vb, bwt, bb, cwt, cb):
    w_spec = pl.BlockSpec((D, D), lambda i: (0, 0))
    b_spec = pl.BlockSpec((1, D), lambda i: (0, 0))
    x_spec = pl.BlockSpec((_NB, D), lambda i: (i, 0))
    return pl.pallas_call(
        _tables_body,
        grid=(N // _NB,),
        in_specs=[x_spec, w_spec, b_spec, w_spec, b_spec, w_spec, b_spec,
                  w_spec, b_spec],
        out_specs=[x_spec, x_spec, x_spec, x_spec],
        out_shape=[jax.ShapeDtypeStruct((N, D), jnp.float32)] * 4,
    )(h_in, uwt, ub, vwt, vb, bwt, bb, cwt, cb)


def _estats_body(e_ref, at, ab, g_ref, sum_ref, ssq_ref):
    pre = (jnp.dot(e_ref[...], at[...], preferred_element_type=jnp.float32)
           + ab[...] + g_ref[...])

    @pl.when(pl.program_id(0) == 0)
    def _():
        sum_ref[...] = jnp.zeros_like(sum_ref)
        ssq_ref[...] = jnp.zeros_like(ssq_ref)

    sum_ref[...] += jnp.sum(pre, axis=0, keepdims=True)
    ssq_ref[...] += jnp.sum(pre * pre, axis=0, keepdims=True)


def _estats(e_in, awt, ab, g):
    w_spec = pl.BlockSpec((D, D), lambda i: (0, 0))
    b_spec = pl.BlockSpec((1, D), lambda i: (0, 0))
    x_spec = pl.BlockSpec((_EB, D), lambda i: (i, 0))
    return pl.pallas_call(
        _estats_body,
        grid=(E // _EB,),
        in_specs=[x_spec, w_spec, b_spec, x_spec],
        out_specs=[b_spec, b_spec],
        out_shape=[jax.ShapeDtypeStruct((1, D), jnp.float32)] * 2,
    )(e_in, awt, ab, g)


def _eout_body(e_ref, at, ab, g_ref, sum_ref, ssq_ref, gam, bet, out_ref):
    pre = (jnp.dot(e_ref[...], at[...], preferred_element_type=jnp.float32)
           + ab[...] + g_ref[...])
    mean = sum_ref[...] * (1.0 / E)
    var = ssq_ref[...] * (1.0 / E) - mean * mean
    inv = lax.rsqrt(var + 1e-5)
    bn = (pre - mean) * inv * gam[...] + bet[...]
    out_ref[...] = e_ref[...] + jnp.maximum(bn, 0.0)


def _eout(e_in, awt, ab, g, esum, essq, gam, bet):
    w_spec = pl.BlockSpec((D, D), lambda i: (0, 0))
    b_spec = pl.BlockSpec((1, D), lambda i: (0, 0))
    x_spec = pl.BlockSpec((_EB, D), lambda i: (i, 0))
    return pl.pallas_call(
        _eout_body,
        grid=(E // _EB,),
        in_specs=[x_spec, w_spec, b_spec, x_spec, b_spec, b_spec, b_spec,
                  b_spec],
        out_specs=x_spec,
        out_shape=jax.ShapeDtypeStruct((E, D), jnp.float32),
    )(e_in, awt, ab, g, esum, essq, gam, bet)


def _hstats_body(uh_ref, a0_ref, a1_ref, sum_ref, ssq_ref):
    pre = uh_ref[...] + a0_ref[...] + a1_ref[...]

    @pl.when(pl.program_id(0) == 0)
    def _():
        sum_ref[...] = jnp.zeros_like(sum_ref)
        ssq_ref[...] = jnp.zeros_like(ssq_ref)

    sum_ref[...] += jnp.sum(pre, axis=0, keepdims=True)
    ssq_ref[...] += jnp.sum(pre * pre, axis=0, keepdims=True)


def _hstats(uh, a0, a1):
    b_spec = pl.BlockSpec((1, D), lambda i: (0, 0))
    x_spec = pl.BlockSpec((_NB, D), lambda i: (i, 0))
    return pl.pallas_call(
        _hstats_body,
        grid=(N // _NB,),
        in_specs=[x_spec, x_spec, x_spec],
        out_specs=[b_spec, b_spec],
        out_shape=[jax.ShapeDtypeStruct((1, D), jnp.float32)] * 2,
    )(uh, a0, a1)


def _hout_body(h_ref, uh_ref, a0_ref, a1_ref, sum_ref, ssq_ref, gam, bet,
               out_ref):
    pre = uh_ref[...] + a0_ref[...] + a1_ref[...]
    mean = sum_ref[...] * (1.0 / N)
    var = ssq_ref[...] * (1.0 / N) - mean * mean
    inv = lax.rsqrt(var + 1e-5)
    bn = (pre - mean) * inv * gam[...] + bet[...]
    out_ref[...] = h_ref[...] + jnp.maximum(bn, 0.0)


def _hout(h_in, uh, a0, a1, hsum, hssq, gam, bet):
    b_spec = pl.BlockSpec((1, D), lambda i: (0, 0))
    x_spec = pl.BlockSpec((_NB, D), lambda i: (i, 0))
    return pl.pallas_call(
        _hout_body,
        grid=(N // _NB,),
        in_specs=[x_spec, x_spec, x_spec, x_spec, b_spec, b_spec, b_spec,
                  b_spec],
        out_specs=x_spec,
        out_shape=jax.ShapeDtypeStruct((N, D), jnp.float32),
    )(h_in, uh, a0, a1, hsum, hssq, gam, bet)


def kernel(h_in, e_in, edge_index, U_w, U_b, V_w, V_b, A_w, A_b, B_w, B_b,
           C_w, C_b, gamma_h, beta_h, gamma_e, beta_e):
    pad = jnp.zeros((2 * K,), jnp.int32)
    idx0 = jnp.concatenate([edge_index[0].astype(jnp.int32), pad])
    idx1 = jnp.concatenate([edge_index[1].astype(jnp.int32), pad])
    ub = U_b.reshape(1, D)
    vb = V_b.reshape(1, D)
    ab = A_b.reshape(1, D)
    bb = B_b.reshape(1, D)
    cb = C_b.reshape(1, D)
    gh = gamma_h.reshape(1, D)
    bh_ = beta_h.reshape(1, D)
    ge = gamma_e.reshape(1, D)
    be = beta_e.reshape(1, D)

    uh, vh, bh, ch = _tables(h_in, U_w.T, ub, V_w.T, vb, B_w.T, bb, C_w.T, cb)

    zeros_n = jnp.zeros((N, D), jnp.float32)
    g, agg = _sc_edges(e_in, idx0, idx1, vh, bh, ch, zeros_n)

    esum, essq = _estats(e_in, A_w.T, ab, g)
    e_out = _eout(e_in, A_w.T, ab, g, esum, essq, ge, be)

    hsum, hssq = _hstats(uh, agg[0], agg[1])
    h_out = _hout(h_in, uh, agg[0], agg[1], hsum, hssq, gh, bh_)
    return (h_out, e_out)


# speedup vs baseline: 1.2935x; 1.0405x over previous
import functools

import jax
import jax.numpy as jnp
import numpy as np
from jax import lax
from jax.experimental import pallas as pl
from jax.experimental.pallas import tpu as pltpu
from jax.experimental.pallas import tpu_sc as plsc

N = 10000
E = 320000
D = 128

NC = 2
NS = 16
NW = NC * NS
EW = E // NW
K = 40
NCHUNK = EW // K
NPAIR = NCHUNK // 2
STRIPE = 624
STRIPE_LAST = N - (NS - 1) * STRIPE


def _worker(c, s):
    return s * NC + c


def _sc_edge_body(e_hbm, i0_hbm, i1_hbm, vh_hbm, bh_hbm, ch_hbm, zeros_hbm,
                  g_hbm, agg_hbm,
                  idx0a, idx1a, idx0b, idx1b,
                  e_a, vh_a, bh_a, ch_a, e_b, vh_b, bh_b, ch_b,
                  agg_sh, sem_a, sem_b, sem_ia, sem_ib, sem_ga, sem_gb):
    c = lax.axis_index("c")
    s = lax.axis_index("s")
    wid = _worker(c, s)
    base0 = wid * EW

    @pl.when(s < NS - 1)
    def _():
        pltpu.sync_copy(zeros_hbm.at[pl.ds(s * STRIPE, STRIPE)],
                        agg_sh.at[pl.ds(s * STRIPE, STRIPE)])

    @pl.when(s == NS - 1)
    def _():
        pltpu.sync_copy(zeros_hbm.at[pl.ds((NS - 1) * STRIPE, STRIPE_LAST)],
                        agg_sh.at[pl.ds((NS - 1) * STRIPE, STRIPE_LAST)])

    plsc.subcore_barrier()

    def fire_idx(ci, i0_v, i1_v, sem):
        base = base0 + ci * K
        pltpu.async_copy(i0_hbm.at[pl.ds(base, K)], i0_v, sem)
        pltpu.async_copy(i1_hbm.at[pl.ds(base, K)], i1_v, sem)

    def wait_idx(i0_v, i1_v, sem):
        pltpu.make_async_copy(i0_hbm.at[pl.ds(0, K)], i0_v, sem).wait()
        pltpu.make_async_copy(i1_hbm.at[pl.ds(0, K)], i1_v, sem).wait()

    def fire4(ci, i0_v, i1_v, e_v, vh_v, bh_v, ch_v, sem):
        pltpu.async_copy(vh_hbm.at[i1_v], vh_v, sem)
        pltpu.async_copy(bh_hbm.at[i0_v], bh_v, sem)
        pltpu.async_copy(ch_hbm.at[i1_v], ch_v, sem)
        pltpu.async_copy(e_hbm.at[pl.ds(base0 + ci * K, K)], e_v, sem)

    def wait4(i0_v, i1_v, e_v, vh_v, bh_v, ch_v, sem):
        pltpu.make_async_copy(vh_hbm.at[i1_v], vh_v, sem).wait()
        pltpu.make_async_copy(bh_hbm.at[i0_v], bh_v, sem).wait()
        pltpu.make_async_copy(ch_hbm.at[i1_v], ch_v, sem).wait()
        pltpu.make_async_copy(e_hbm.at[pl.ds(0, K)], e_v, sem).wait()

    def compute(e_v, vh_v, bh_v, ch_v):
        def row_body(r, rc):
            for cc in range(D // 16):
                sl = pl.ds(cc * 16, 16)
                x = e_v[r, sl]
                e_v[r, sl] = vh_v[r, sl] / (1.0 + jnp.exp(-x))
                vh_v[r, sl] = bh_v[r, sl] + ch_v[r, sl]
            return rc

        lax.fori_loop(0, K, row_body, 0)

    def fire_g(ci, g_v, sem):
        pltpu.async_copy(g_v, g_hbm.at[pl.ds(base0 + ci * K, K)], sem)

    def wait_g(g_v, sem):
        pltpu.make_async_copy(g_v, g_hbm.at[pl.ds(0, K)], sem).wait()

    pltpu.sync_copy(i0_hbm.at[pl.ds(base0, K)], idx0a)
    pltpu.sync_copy(i1_hbm.at[pl.ds(base0, K)], idx1a)
    fire4(0, idx0a, idx1a, e_a, vh_a, bh_a, ch_a, sem_a)
    fire_idx(1, idx0b, idx1b, sem_ib)

    def pair_body(pi, carry):
        c0 = 2 * pi
        c1 = c0 + 1
        wait4(idx0a, idx1a, e_a, vh_a, bh_a, ch_a, sem_a)
        wait_idx(idx0b, idx1b, sem_ib)

        @pl.when(pi > 0)
        def _():
            wait_g(vh_b, sem_gb)

        fire4(c1, idx0b, idx1b, e_b, vh_b, bh_b, ch_b, sem_b)
        compute(e_a, vh_a, bh_a, ch_a)
        fire_g(c0, vh_a, sem_ga)
        pltpu.sync_copy(e_a, agg_sh.at[idx0a], add=True)
        fire_idx(c0 + 2, idx0a, idx1a, sem_ia)
        wait4(idx0b, idx1b, e_b, vh_b, bh_b, ch_b, sem_b)
        wait_idx(idx0a, idx1a, sem_ia)
        wait_g(vh_a, sem_ga)

        @pl.when(pi < NPAIR - 1)
        def _():
            fire4(c0 + 2, idx0a, idx1a, e_a, vh_a, bh_a, ch_a, sem_a)

        compute(e_b, vh_b, bh_b, ch_b)
        fire_g(c1, vh_b, sem_gb)
        pltpu.sync_copy(e_b, agg_sh.at[idx0b], add=True)
        fire_idx(c0 + 3, idx0b, idx1b, sem_ib)
        return carry

    lax.fori_loop(0, NPAIR, pair_body, 0)
    wait_idx(idx0b, idx1b, sem_ib)
    wait_g(vh_b, sem_gb)
    plsc.subcore_barrier()

    @pl.when(s < NS - 1)
    def _():
        pltpu.sync_copy(agg_sh.at[pl.ds(s * STRIPE, STRIPE)],
                        agg_hbm.at[c, pl.ds(s * STRIPE, STRIPE)])

    @pl.when(s == NS - 1)
    def _():
        pltpu.sync_copy(agg_sh.at[pl.ds((NS - 1) * STRIPE, STRIPE_LAST)],
                        agg_hbm.at[c, pl.ds((NS - 1) * STRIPE, STRIPE_LAST)])


def _sc_edges(e_in, idx0, idx1, vh, bh, ch, zeros_n):
    mesh = plsc.VectorSubcoreMesh(core_axis_name="c", subcore_axis_name="s",
                                  num_cores=NC, num_subcores=NS)
    return pl.kernel(
        _sc_edge_body,
        out_type=(jax.ShapeDtypeStruct((E, D), jnp.float32),
                  jax.ShapeDtypeStruct((NC, N, D), jnp.float32)),
        mesh=mesh,
        scratch_types=(
            [pltpu.VMEM((K,), jnp.int32)] * 4
            + [pltpu.VMEM((K, D), jnp.float32)] * 8
            + [pltpu.VMEM_SHARED((N, D), jnp.float32)]
            + [pltpu.SemaphoreType.DMA] * 6
        ),
    )(e_in, idx0, idx1, vh, bh, ch, zeros_n)


_NB = 1000
_EB = 8000


def _tables_body(h_ref, uw, ub, vw, vb, bw, bb, cw, cb,
                 uh_ref, vh_ref, bh_ref, ch_ref):
    h = h_ref[...]
    uh_ref[...] = jnp.dot(h, uw[...], preferred_element_type=jnp.float32) + ub[...]
    vh_ref[...] = jnp.dot(h, vw[...], preferred_element_type=jnp.float32) + vb[...]
    bh_ref[...] = jnp.dot(h, bw[...], preferred_element_type=jnp.float32) + bb[...]
    ch_ref[...] = jnp.dot(h, cw[...], preferred_element_type=jnp.float32) + cb[...]


def _tables(h_in, uwt, ub, vwt, vb, bwt, bb, cwt, cb):
    w_spec = pl.BlockSpec((D, D), lambda i: (0, 0))
    b_spec = pl.BlockSpec((1, D), lambda i: (0, 0))
    x_spec = pl.BlockSpec((_NB, D), lambda i: (i, 0))
    return pl.pallas_call(
        _tables_body,
        grid=(N // _NB,),
        in_specs=[x_spec, w_spec, b_spec, w_spec, b_spec, w_spec, b_spec,
                  w_spec, b_spec],
        out_specs=[x_spec, x_spec, x_spec, x_spec],
        out_shape=[jax.ShapeDtypeStruct((N, D), jnp.float32)] * 4,
    )(h_in, uwt, ub, vwt, vb, bwt, bb, cwt, cb)


def _estats_body(e_ref, at, ab, g_ref, sum_ref, ssq_ref):
    pre = (jnp.dot(e_ref[...], at[...], preferred_element_type=jnp.float32)
           + ab[...] + g_ref[...])

    @pl.when(pl.program_id(0) == 0)
    def _():
        sum_ref[...] = jnp.zeros_like(sum_ref)
        ssq_ref[...] = jnp.zeros_like(ssq_ref)

    sum_ref[...] += jnp.sum(pre, axis=0, keepdims=True)
    ssq_ref[...] += jnp.sum(pre * pre, axis=0, keepdims=True)


def _estats(e_in, awt, ab, g):
    w_spec = pl.BlockSpec((D, D), lambda i: (0, 0))
    b_spec = pl.BlockSpec((1, D), lambda i: (0, 0))
    x_spec = pl.BlockSpec((_EB, D), lambda i: (i, 0))
    return pl.pallas_call(
        _estats_body,
        grid=(E // _EB,),
        in_specs=[x_spec, w_spec, b_spec, x_spec],
        out_specs=[b_spec, b_spec],
        out_shape=[jax.ShapeDtypeStruct((1, D), jnp.float32)] * 2,
    )(e_in, awt, ab, g)


def _eout_body(e_ref, at, ab, g_ref, sum_ref, ssq_ref, gam, bet, out_ref):
    pre = (jnp.dot(e_ref[...], at[...], preferred_element_type=jnp.float32)
           + ab[...] + g_ref[...])
    mean = sum_ref[...] * (1.0 / E)
    var = ssq_ref[...] * (1.0 / E) - mean * mean
    inv = lax.rsqrt(var + 1e-5)
    bn = (pre - mean) * inv * gam[...] + bet[...]
    out_ref[...] = e_ref[...] + jnp.maximum(bn, 0.0)


def _eout(e_in, awt, ab, g, esum, essq, gam, bet):
    w_spec = pl.BlockSpec((D, D), lambda i: (0, 0))
    b_spec = pl.BlockSpec((1, D), lambda i: (0, 0))
    x_spec = pl.BlockSpec((_EB, D), lambda i: (i, 0))
    return pl.pallas_call(
        _eout_body,
        grid=(E // _EB,),
        in_specs=[x_spec, w_spec, b_spec, x_spec, b_spec, b_spec, b_spec,
                  b_spec],
        out_specs=x_spec,
        out_shape=jax.ShapeDtypeStruct((E, D), jnp.float32),
    )(e_in, awt, ab, g, esum, essq, gam, bet)


def _hstats_body(uh_ref, a0_ref, a1_ref, sum_ref, ssq_ref):
    pre = uh_ref[...] + a0_ref[...] + a1_ref[...]

    @pl.when(pl.program_id(0) == 0)
    def _():
        sum_ref[...] = jnp.zeros_like(sum_ref)
        ssq_ref[...] = jnp.zeros_like(ssq_ref)

    sum_ref[...] += jnp.sum(pre, axis=0, keepdims=True)
    ssq_ref[...] += jnp.sum(pre * pre, axis=0, keepdims=True)


def _hstats(uh, a0, a1):
    b_spec = pl.BlockSpec((1, D), lambda i: (0, 0))
    x_spec = pl.BlockSpec((_NB, D), lambda i: (i, 0))
    return pl.pallas_call(
        _hstats_body,
        grid=(N // _NB,),
        in_specs=[x_spec, x_spec, x_spec],
        out_specs=[b_spec, b_spec],
        out_shape=[jax.ShapeDtypeStruct((1, D), jnp.float32)] * 2,
    )(uh, a0, a1)


def _hout_body(h_ref, uh_ref, a0_ref, a1_ref, sum_ref, ssq_ref, gam, bet,
               out_ref):
    pre = uh_ref[...] + a0_ref[...] + a1_ref[...]
    mean = sum_ref[...] * (1.0 / N)
    var = ssq_ref[...] * (1.0 / N) - mean * mean
    inv = lax.rsqrt(var + 1e-5)
    bn = (pre - mean) * inv * gam[...] + bet[...]
    out_ref[...] = h_ref[...] + jnp.maximum(bn, 0.0)


def _hout(h_in, uh, a0, a1, hsum, hssq, gam, bet):
    b_spec = pl.BlockSpec((1, D), lambda i: (0, 0))
    x_spec = pl.BlockSpec((_NB, D), lambda i: (i, 0))
    return pl.pallas_call(
        _hout_body,
        grid=(N // _NB,),
        in_specs=[x_spec, x_spec, x_spec, x_spec, b_spec, b_spec, b_spec,
                  b_spec],
        out_specs=x_spec,
        out_shape=jax.ShapeDtypeStruct((N, D), jnp.float32),
    )(h_in, uh, a0, a1, hsum, hssq, gam, bet)


def kernel(h_in, e_in, edge_index, U_w, U_b, V_w, V_b, A_w, A_b, B_w, B_b,
           C_w, C_b, gamma_h, beta_h, gamma_e, beta_e):
    pad = jnp.zeros((2 * K,), jnp.int32)
    idx0 = jnp.concatenate([edge_index[0].astype(jnp.int32), pad])
    idx1 = jnp.concatenate([edge_index[1].astype(jnp.int32), pad])
    ub = U_b.reshape(1, D)
    vb = V_b.reshape(1, D)
    ab = A_b.reshape(1, D)
    bb = B_b.reshape(1, D)
    cb = C_b.reshape(1, D)
    gh = gamma_h.reshape(1, D)
    bh_ = beta_h.reshape(1, D)
    ge = gamma_e.reshape(1, D)
    be = beta_e.reshape(1, D)

    uh, vh, bh, ch = _tables(h_in, U_w.T, ub, V_w.T, vb, B_w.T, bb, C_w.T, cb)

    zeros_n = jnp.zeros((N, D), jnp.float32)
    g, agg = _sc_edges(e_in, idx0, idx1, vh, bh, ch, zeros_n)

    esum, essq = _estats(e_in, A_w.T, ab, g)
    e_out = _eout(e_in, A_w.T, ab, g, esum, essq, ge, be)

    hsum, hssq = _hstats(uh, agg[0], agg[1])
    h_out = _hout(h_in, uh, agg[0], agg[1], hsum, hssq, gh, bh_)
    return (h_out, e_out)
